# baseline (speedup 1.0000x reference)
import functools

import jax
import jax.numpy as jnp
from jax import lax
from jax.experimental import pallas as pl
from jax.experimental.pallas import tpu as pltpu
from jax.experimental.pallas import tpu_sc as plsc

N_NODES = 100000
NP = 100352
NPT = NP // 16
N_EDGES = 6400000
NROWS = N_EDGES // 128
NC, NS = 2, 16
NW = NC * NS
OCT_TOT = NROWS // 8
OCT_PT = OCT_TOT // NW
OCT_REM = OCT_TOT - OCT_PT * NW
BLK = 24
NBLK = (OCT_PT * 8) // BLK

_mesh = plsc.VectorSubcoreMesh(core_axis_name="c", subcore_axis_name="s")


def _zero_shared_slab(bounce_v, acc_sh, s):
    def zbody(i, _):
        bounce_v[pl.ds(i * 16, 16)] = jnp.zeros((16,), jnp.float32)
        return 0

    lax.fori_loop(0, NPT // 16, zbody, 0)
    pltpu.sync_copy(bounce_v, acc_sh.at[pl.ds(s * NPT, NPT)])


def _readout_shared_slab(acc_sh, bounce_v, out_hbm, c, s):
    pltpu.sync_copy(acc_sh.at[pl.ds(s * NPT, NPT)], bounce_v)
    pltpu.sync_copy(bounce_v, out_hbm.at[pl.ds(c * NP + s * NPT, NPT)])


@functools.partial(
    pl.kernel,
    out_type=jax.ShapeDtypeStruct((NC * NP,), jnp.float32),
    mesh=_mesh,
    scratch_types=[
        pltpu.VMEM((BLK * 128,), jnp.int32),
        pltpu.VMEM((128,), jnp.float32),
        pltpu.VMEM((NPT,), jnp.float32),
        pltpu.VMEM_SHARED((NP,), jnp.float32),
        pltpu.SemaphoreType.DMA,
    ],
    compiler_params=pltpu.CompilerParams(use_tc_tiling_on_sc=False),
)
def _deg_kernel(edge_ref, deg_out, idx_v, ones_v, bounce_v, acc_sh, sem):
    c = lax.axis_index("c")
    s = lax.axis_index("s")
    wid = c * NS + s
    for i in range(8):
        ones_v[pl.ds(i * 16, 16)] = jnp.ones((16,), jnp.float32)
    _zero_shared_slab(bounce_v, acc_sh, s)
    plsc.subcore_barrier()

    row_base = (wid * OCT_PT + jnp.minimum(wid, OCT_REM)) * 8

    def blk(g, _):
        pltpu.sync_copy(
            edge_ref.at[1, pl.ds((row_base + g * BLK) * 128, BLK * 128)],
            idx_v)
        hs = [
            pltpu.async_copy(ones_v, acc_sh.at[idx_v.at[pl.ds(j * 128, 128)]],
                             sem, add=True)
            for j in range(BLK)
        ]
        for h in hs:
            h.wait()
        return 0

    lax.fori_loop(0, NBLK, blk, 0)

    @pl.when(wid < OCT_REM)
    def _extra_octet():
        pltpu.sync_copy(
            edge_ref.at[1, pl.ds((row_base + NBLK * BLK) * 128, 8 * 128)],
            idx_v.at[pl.ds(0, 8 * 128)])
        hs = [
            pltpu.async_copy(ones_v, acc_sh.at[idx_v.at[pl.ds(j * 128, 128)]],
                             sem, add=True)
            for j in range(8)
        ]
        for h in hs:
            h.wait()

    plsc.subcore_barrier()
    _readout_shared_slab(acc_sh, bounce_v, deg_out, c, s)


@functools.partial(
    pl.kernel,
    out_type=jax.ShapeDtypeStruct((NC * NP,), jnp.float32),
    mesh=_mesh,
    scratch_types=[
        pltpu.VMEM((N_NODES,), jnp.float32),
        pltpu.VMEM((BLK * 128,), jnp.int32),
        pltpu.VMEM((BLK * 128,), jnp.int32),
        pltpu.VMEM((BLK * 128,), jnp.float32),
        pltpu.VMEM((BLK * 128,), jnp.int32),
        pltpu.VMEM((BLK * 128,), jnp.int32),
        pltpu.VMEM((BLK * 128,), jnp.float32),
        pltpu.VMEM((NPT,), jnp.float32),
        pltpu.VMEM_SHARED((NP,), jnp.float32),
        pltpu.SemaphoreType.DMA,
        pltpu.SemaphoreType.DMA,
        pltpu.SemaphoreType.DMA,
        pltpu.SemaphoreType.DMA,
    ],
    compiler_params=pltpu.CompilerParams(
        needs_layout_passes=False, use_tc_tiling_on_sc=False),
)
def _msg_kernel(edge_ref, g_ref, out_hbm, g_v, row0, col0, vals0, row1, col1,
                vals1, bounce_v, acc_sh, ld0, ld1, sc0, sc1):
    c = lax.axis_index("c")
    s = lax.axis_index("s")
    wid = c * NS + s
    row_base = (wid * OCT_PT + jnp.minimum(wid, OCT_REM)) * 8

    def load_start(rowX, colX, semX, blk_idx):
        off = (row_base + blk_idx * BLK) * 128
        pltpu.async_copy(edge_ref.at[0, pl.ds(off, BLK * 128)], rowX, semX)
        pltpu.async_copy(edge_ref.at[1, pl.ds(off, BLK * 128)], colX, semX)

    def load_wait(rowX, colX, semX):
        pltpu.make_async_copy(edge_ref.at[0, pl.ds(0, BLK * 128)], rowX,
                              semX).wait()
        pltpu.make_async_copy(edge_ref.at[0, pl.ds(0, BLK * 128)], colX,
                              semX).wait()

    def compute(rowX, valsX, n=BLK):
        for m in range(n * 8):
            iv = rowX[pl.ds(m * 16, 16)]
            valsX[pl.ds(m * 16, 16)] = plsc.load_gather(g_v, [iv])

    def fire(colX, valsX, semX, n=BLK):
        for j in range(n):
            pltpu.async_copy(valsX.at[pl.ds(j * 128, 128)],
                             acc_sh.at[colX.at[pl.ds(j * 128, 128)]],
                             semX, add=True)

    def drain(semX, n=BLK):
        for _ in range(n):
            pltpu.make_async_copy(g_ref.at[pl.ds(0, 128)],
                                  bounce_v.at[pl.ds(0, 128)], semX).wait()

    load_start(row0, col0, ld0, 0)
    _zero_shared_slab(bounce_v, acc_sh, s)
    pltpu.sync_copy(g_ref, g_v)
    plsc.subcore_barrier()

    def pair(g, _):
        e0 = 2 * g
        load_wait(row0, col0, ld0)
        compute(row0, vals0)
        fire(col0, vals0, sc0)
        @pl.when(g >= 1)
        def _():
            drain(sc1)
        load_start(row1, col1, ld1, e0 + 1)
        load_wait(row1, col1, ld1)
        compute(row1, vals1)
        fire(col1, vals1, sc1)
        drain(sc0)
        load_start(row0, col0, ld0, e0 + 2)
        return 0

    lax.fori_loop(0, (NBLK - 1) // 2, pair, 0)

    load_wait(row0, col0, ld0)
    compute(row0, vals0)
    fire(col0, vals0, sc0)
    drain(sc1)

    @pl.when(wid < OCT_REM)
    def _extra_octet():
        toff = (row_base + NBLK * BLK) * 128
        pltpu.sync_copy(edge_ref.at[0, pl.ds(toff, 8 * 128)],
                        row1.at[pl.ds(0, 8 * 128)])
        pltpu.sync_copy(edge_ref.at[1, pl.ds(toff, 8 * 128)],
                        col1.at[pl.ds(0, 8 * 128)])
        compute(row1, vals1, n=8)
        fire(col1, vals1, sc1, n=8)
        drain(sc1, n=8)

    drain(sc0)
    plsc.subcore_barrier()
    _readout_shared_slab(acc_sh, bounce_v, out_hbm, c, s)


def _mid_body(deg_ref, x_ref, dinv_ref, g_ref):
    d = deg_ref[0] + deg_ref[1] + 1.0
    dinv = lax.rsqrt(d)
    dinv_ref[...] = dinv
    g_ref[...] = dinv * x_ref[...]


def _fin_body(o_ref, dinv_ref, g_ref, w_ref, b_ref, out_ref):
    ssum = o_ref[0] + o_ref[1] + g_ref[...]
    z = w_ref[0, 0] * (dinv_ref[...] * ssum) + b_ref[0, 0]
    out_ref[...] = jax.nn.sigmoid(z)


def kernel(x, edge_index, W, b):
    e = edge_index.astype(jnp.int32)
    xp = jnp.pad(x.reshape(-1), (0, NP - N_NODES)).reshape(784, 128)

    deg_parts = _deg_kernel(e)

    dinv, g = pl.pallas_call(
        _mid_body,
        out_shape=[
            jax.ShapeDtypeStruct((784, 128), jnp.float32),
            jax.ShapeDtypeStruct((784, 128), jnp.float32),
        ],
    )(deg_parts.reshape(2, 784, 128), xp)

    out_parts = _msg_kernel(e, g.reshape(NP)[:N_NODES])

    fin = pl.pallas_call(
        _fin_body,
        out_shape=jax.ShapeDtypeStruct((784, 128), jnp.float32),
    )(
        out_parts.reshape(2, 784, 128),
        dinv,
        g,
        W.astype(jnp.float32).reshape(1, 1),
        b.astype(jnp.float32).reshape(1, 1),
    )

    return fin.reshape(NP, 1)[:N_NODES]


# speedup vs baseline: 1.0661x; 1.0661x over previous
import functools

import jax
import jax.numpy as jnp
from jax import lax
from jax.experimental import pallas as pl
from jax.experimental.pallas import tpu as pltpu
from jax.experimental.pallas import tpu_sc as plsc

N_NODES = 100000
NP = 100352
NPT = NP // 16
N_EDGES = 6400000
NROWS = N_EDGES // 128
NC, NS = 2, 16
NW = NC * NS
OCT_TOT = NROWS // 8
OCT_PT = OCT_TOT // NW
OCT_REM = OCT_TOT - OCT_PT * NW
BLK = 24
NBLK = (OCT_PT * 8) // BLK

_mesh = plsc.VectorSubcoreMesh(core_axis_name="c", subcore_axis_name="s")


def _zero_shared_slab(bounce_v, acc_sh, s):
    def zbody(i, _):
        bounce_v[pl.ds(i * 16, 16)] = jnp.zeros((16,), jnp.float32)
        return 0

    lax.fori_loop(0, NPT // 16, zbody, 0)
    pltpu.sync_copy(bounce_v, acc_sh.at[pl.ds(s * NPT, NPT)])


def _readout_shared_slab(acc_sh, bounce_v, out_hbm, c, s):
    pltpu.sync_copy(acc_sh.at[pl.ds(s * NPT, NPT)], bounce_v)
    pltpu.sync_copy(bounce_v, out_hbm.at[pl.ds(c * NP + s * NPT, NPT)])


@functools.partial(
    pl.kernel,
    out_type=jax.ShapeDtypeStruct((NW * NP,), jnp.float32),
    mesh=_mesh,
    scratch_types=[
        pltpu.VMEM((NP,), jnp.float32),
        pltpu.VMEM((BLK * 128,), jnp.int32),
        pltpu.VMEM((BLK * 128,), jnp.int32),
        pltpu.SemaphoreType.DMA,
        pltpu.SemaphoreType.DMA,
    ],
    compiler_params=pltpu.CompilerParams(
        needs_layout_passes=False, use_tc_tiling_on_sc=False),
)
def _deg_kernel(edge_ref, deg_out, hist_v, idx0, idx1, ld0, ld1):
    c = lax.axis_index("c")
    s = lax.axis_index("s")
    wid = c * NS + s
    row_base = (wid * OCT_PT + jnp.minimum(wid, OCT_REM)) * 8
    ones16 = jnp.ones((16,), jnp.float32)

    def load_start(idxX, semX, blk_idx):
        pltpu.async_copy(
            edge_ref.at[1, pl.ds((row_base + blk_idx * BLK) * 128, BLK * 128)],
            idxX, semX)

    def load_wait(idxX, semX):
        pltpu.make_async_copy(edge_ref.at[1, pl.ds(0, BLK * 128)], idxX,
                              semX).wait()

    def process(idxX, n=BLK):
        for m in range(n * 8):
            iv = idxX[pl.ds(m * 16, 16)]
            plsc.addupdate_scatter(hist_v, [iv], ones16)

    load_start(idx0, ld0, 0)

    def zbody(i, _):
        for u in range(8):
            hist_v[pl.ds(i * 128 + u * 16, 16)] = jnp.zeros((16,), jnp.float32)
        return 0

    lax.fori_loop(0, NP // 128, zbody, 0)

    def pair(g, _):
        load_wait(idx0, ld0)
        load_start(idx1, ld1, 2 * g + 1)
        process(idx0)
        load_wait(idx1, ld1)
        load_start(idx0, ld0, 2 * g + 2)
        process(idx1)
        return 0

    lax.fori_loop(0, (NBLK - 1) // 2, pair, 0)

    load_wait(idx0, ld0)
    process(idx0)

    @pl.when(wid < OCT_REM)
    def _extra_octet():
        pltpu.sync_copy(
            edge_ref.at[1, pl.ds((row_base + NBLK * BLK) * 128, 8 * 128)],
            idx1.at[pl.ds(0, 8 * 128)])
        process(idx1, n=8)

    pltpu.sync_copy(hist_v, deg_out.at[pl.ds(wid * NP, NP)])


@functools.partial(
    pl.kernel,
    out_type=jax.ShapeDtypeStruct((NC * NP,), jnp.float32),
    mesh=_mesh,
    scratch_types=[
        pltpu.VMEM((N_NODES,), jnp.float32),
        pltpu.VMEM((BLK * 128,), jnp.int32),
        pltpu.VMEM((BLK * 128,), jnp.int32),
        pltpu.VMEM((BLK * 128,), jnp.float32),
        pltpu.VMEM((BLK * 128,), jnp.int32),
        pltpu.VMEM((BLK * 128,), jnp.int32),
        pltpu.VMEM((BLK * 128,), jnp.float32),
        pltpu.VMEM((NPT,), jnp.float32),
        pltpu.VMEM_SHARED((NP,), jnp.float32),
        pltpu.SemaphoreType.DMA,
        pltpu.SemaphoreType.DMA,
        pltpu.SemaphoreType.DMA,
        pltpu.SemaphoreType.DMA,
    ],
    compiler_params=pltpu.CompilerParams(
        needs_layout_passes=False, use_tc_tiling_on_sc=False),
)
def _msg_kernel(edge_ref, g_ref, out_hbm, g_v, row0, col0, vals0, row1, col1,
                vals1, bounce_v, acc_sh, ld0, ld1, sc0, sc1):
    c = lax.axis_index("c")
    s = lax.axis_index("s")
    wid = c * NS + s
    row_base = (wid * OCT_PT + jnp.minimum(wid, OCT_REM)) * 8

    def load_start(rowX, colX, semX, blk_idx):
        off = (row_base + blk_idx * BLK) * 128
        pltpu.async_copy(edge_ref.at[0, pl.ds(off, BLK * 128)], rowX, semX)
        pltpu.async_copy(edge_ref.at[1, pl.ds(off, BLK * 128)], colX, semX)

    def load_wait(rowX, colX, semX):
        pltpu.make_async_copy(edge_ref.at[0, pl.ds(0, BLK * 128)], rowX,
                              semX).wait()
        pltpu.make_async_copy(edge_ref.at[0, pl.ds(0, BLK * 128)], colX,
                              semX).wait()

    def compute(rowX, valsX, n=BLK):
        for m in range(n * 8):
            iv = rowX[pl.ds(m * 16, 16)]
            valsX[pl.ds(m * 16, 16)] = plsc.load_gather(g_v, [iv])

    def fire(colX, valsX, semX, n=BLK):
        for j in range(n):
            pltpu.async_copy(valsX.at[pl.ds(j * 128, 128)],
                             acc_sh.at[colX.at[pl.ds(j * 128, 128)]],
                             semX, add=True)

    def drain(semX, n=BLK):
        for _ in range(n):
            pltpu.make_async_copy(g_ref.at[pl.ds(0, 128)],
                                  bounce_v.at[pl.ds(0, 128)], semX).wait()

    load_start(row0, col0, ld0, 0)
    _zero_shared_slab(bounce_v, acc_sh, s)
    pltpu.sync_copy(g_ref, g_v)
    plsc.subcore_barrier()

    def pair(g, _):
        e0 = 2 * g
        load_wait(row0, col0, ld0)
        compute(row0, vals0)
        fire(col0, vals0, sc0)
        @pl.when(g >= 1)
        def _():
            drain(sc1)
        load_start(row1, col1, ld1, e0 + 1)
        load_wait(row1, col1, ld1)
        compute(row1, vals1)
        fire(col1, vals1, sc1)
        drain(sc0)
        load_start(row0, col0, ld0, e0 + 2)
        return 0

    lax.fori_loop(0, (NBLK - 1) // 2, pair, 0)

    load_wait(row0, col0, ld0)
    compute(row0, vals0)
    fire(col0, vals0, sc0)
    drain(sc1)

    @pl.when(wid < OCT_REM)
    def _extra_octet():
        toff = (row_base + NBLK * BLK) * 128
        pltpu.sync_copy(edge_ref.at[0, pl.ds(toff, 8 * 128)],
                        row1.at[pl.ds(0, 8 * 128)])
        pltpu.sync_copy(edge_ref.at[1, pl.ds(toff, 8 * 128)],
                        col1.at[pl.ds(0, 8 * 128)])
        compute(row1, vals1, n=8)
        fire(col1, vals1, sc1, n=8)
        drain(sc1, n=8)

    drain(sc0)
    plsc.subcore_barrier()
    _readout_shared_slab(acc_sh, bounce_v, out_hbm, c, s)


def _mid_body(deg_ref, x_ref, dinv_ref, g_ref):
    d = jnp.sum(deg_ref[...], axis=0) + 1.0
    dinv = lax.rsqrt(d)
    dinv_ref[...] = dinv
    g_ref[...] = dinv * x_ref[...]


def _fin_body(o_ref, dinv_ref, g_ref, w_ref, b_ref, out_ref):
    ssum = o_ref[0] + o_ref[1] + g_ref[...]
    z = w_ref[0, 0] * (dinv_ref[...] * ssum) + b_ref[0, 0]
    out_ref[...] = jax.nn.sigmoid(z)


def kernel(x, edge_index, W, b):
    e = edge_index.astype(jnp.int32)
    xp = jnp.pad(x.reshape(-1), (0, NP - N_NODES)).reshape(784, 128)

    deg_parts = _deg_kernel(e)

    dinv, g = pl.pallas_call(
        _mid_body,
        out_shape=[
            jax.ShapeDtypeStruct((784, 128), jnp.float32),
            jax.ShapeDtypeStruct((784, 128), jnp.float32),
        ],
    )(deg_parts.reshape(NW, 784, 128), xp)

    out_parts = _msg_kernel(e, g.reshape(NP)[:N_NODES])

    fin = pl.pallas_call(
        _fin_body,
        out_shape=jax.ShapeDtypeStruct((784, 128), jnp.float32),
    )(
        out_parts.reshape(2, 784, 128),
        dinv,
        g,
        W.astype(jnp.float32).reshape(1, 1),
        b.astype(jnp.float32).reshape(1, 1),
    )

    return fin.reshape(NP, 1)[:N_NODES]


# speedup vs baseline: 1.0996x; 1.0314x over previous
import functools

import jax
import jax.numpy as jnp
from jax import lax
from jax.experimental import pallas as pl
from jax.experimental.pallas import tpu as pltpu
from jax.experimental.pallas import tpu_sc as plsc

N_NODES = 100000
NP = 100352
NPT = NP // 16
N_EDGES = 6400000
NROWS = N_EDGES // 128
NC, NS = 2, 16
NW = NC * NS
OCT_TOT = NROWS // 8
OCT_PT = OCT_TOT // NW
OCT_REM = OCT_TOT - OCT_PT * NW
BLK = 24
NBLK = (OCT_PT * 8) // BLK

_mesh = plsc.VectorSubcoreMesh(core_axis_name="c", subcore_axis_name="s")


def _zero_shared_slab(bounce_v, acc_sh, s):
    def zbody(i, _):
        bounce_v[pl.ds(i * 16, 16)] = jnp.zeros((16,), jnp.float32)
        return 0

    lax.fori_loop(0, NPT // 16, zbody, 0)
    pltpu.sync_copy(bounce_v, acc_sh.at[pl.ds(s * NPT, NPT)])


def _readout_shared_slab(acc_sh, bounce_v, out_hbm, c, s):
    pltpu.sync_copy(acc_sh.at[pl.ds(s * NPT, NPT)], bounce_v)
    pltpu.sync_copy(bounce_v, out_hbm.at[pl.ds(c * NP + s * NPT, NPT)])


@functools.partial(
    pl.kernel,
    out_type=jax.ShapeDtypeStruct((NW * NP,), jnp.float32),
    mesh=_mesh,
    scratch_types=[
        pltpu.VMEM((NP,), jnp.float32),
        pltpu.VMEM((BLK * 128,), jnp.int32),
        pltpu.VMEM((BLK * 128,), jnp.int32),
        pltpu.SemaphoreType.DMA,
        pltpu.SemaphoreType.DMA,
    ],
    compiler_params=pltpu.CompilerParams(
        needs_layout_passes=False, use_tc_tiling_on_sc=False),
)
def _deg_kernel(edge_ref, deg_out, hist_v, idx0, idx1, ld0, ld1):
    c = lax.axis_index("c")
    s = lax.axis_index("s")
    wid = c * NS + s
    row_base = (wid * OCT_PT + jnp.minimum(wid, OCT_REM)) * 8
    ones16 = jnp.ones((16,), jnp.float32)

    def load_start(idxX, semX, blk_idx):
        pltpu.async_copy(
            edge_ref.at[1, pl.ds((row_base + blk_idx * BLK) * 128, BLK * 128)],
            idxX, semX)

    def load_wait(idxX, semX):
        pltpu.make_async_copy(edge_ref.at[1, pl.ds(0, BLK * 128)], idxX,
                              semX).wait()

    def process(idxX, n=BLK):
        for m in range(n * 8):
            iv = idxX[pl.ds(m * 16, 16)]
            plsc.addupdate_scatter(hist_v, [iv], ones16)

    load_start(idx0, ld0, 0)

    def zbody(i, _):
        for u in range(8):
            hist_v[pl.ds(i * 128 + u * 16, 16)] = jnp.zeros((16,), jnp.float32)
        return 0

    lax.fori_loop(0, NP // 128, zbody, 0)

    def pair(g, _):
        load_wait(idx0, ld0)
        load_start(idx1, ld1, 2 * g + 1)
        process(idx0)
        load_wait(idx1, ld1)
        load_start(idx0, ld0, 2 * g + 2)
        process(idx1)
        return 0

    lax.fori_loop(0, (NBLK - 1) // 2, pair, 0)

    load_wait(idx0, ld0)
    process(idx0)

    @pl.when(wid < OCT_REM)
    def _extra_octet():
        pltpu.sync_copy(
            edge_ref.at[1, pl.ds((row_base + NBLK * BLK) * 128, 8 * 128)],
            idx1.at[pl.ds(0, 8 * 128)])
        process(idx1, n=8)

    pltpu.sync_copy(hist_v, deg_out.at[pl.ds(wid * NP, NP)])


@functools.partial(
    pl.kernel,
    out_type=jax.ShapeDtypeStruct((NC * NP,), jnp.float32),
    mesh=_mesh,
    scratch_types=[
        pltpu.VMEM((N_NODES,), jnp.float32),
        pltpu.VMEM((BLK * 128,), jnp.int32),
        pltpu.VMEM((BLK * 128,), jnp.int32),
        pltpu.VMEM((BLK * 128,), jnp.float32),
        pltpu.VMEM((BLK * 128,), jnp.int32),
        pltpu.VMEM((BLK * 128,), jnp.int32),
        pltpu.VMEM((BLK * 128,), jnp.float32),
        pltpu.VMEM((NPT,), jnp.float32),
        pltpu.VMEM_SHARED((NP,), jnp.float32),
        pltpu.SemaphoreType.DMA,
        pltpu.SemaphoreType.DMA,
        pltpu.SemaphoreType.DMA,
        pltpu.SemaphoreType.DMA,
    ],
    compiler_params=pltpu.CompilerParams(
        needs_layout_passes=False, use_tc_tiling_on_sc=False),
)
def _msg_kernel(edge_ref, g_ref, out_hbm, g_v, row0, col0, vals0, row1, col1,
                vals1, bounce_v, acc_sh, ld0, ld1, sc0, sc1):
    c = lax.axis_index("c")
    s = lax.axis_index("s")
    wid = c * NS + s
    row_base = (wid * OCT_PT + jnp.minimum(wid, OCT_REM)) * 8

    def load_start_row(rowX, semX, blk_idx):
        off = (row_base + blk_idx * BLK) * 128
        pltpu.async_copy(edge_ref.at[0, pl.ds(off, BLK * 128)], rowX, semX)

    def load_start_col(colX, semX, blk_idx):
        off = (row_base + blk_idx * BLK) * 128
        pltpu.async_copy(edge_ref.at[1, pl.ds(off, BLK * 128)], colX, semX)

    def load_start(rowX, colX, semX, blk_idx):
        load_start_row(rowX, semX, blk_idx)
        load_start_col(colX, semX, blk_idx)

    def load_wait(rowX, colX, semX):
        pltpu.make_async_copy(edge_ref.at[0, pl.ds(0, BLK * 128)], rowX,
                              semX).wait()
        pltpu.make_async_copy(edge_ref.at[0, pl.ds(0, BLK * 128)], colX,
                              semX).wait()

    def compute(rowX, valsX, n=BLK):
        for m in range(n * 8):
            iv = rowX[pl.ds(m * 16, 16)]
            valsX[pl.ds(m * 16, 16)] = plsc.load_gather(g_v, [iv])

    def fire(colX, valsX, semX, n=BLK):
        for j in range(n):
            pltpu.async_copy(valsX.at[pl.ds(j * 128, 128)],
                             acc_sh.at[colX.at[pl.ds(j * 128, 128)]],
                             semX, add=True)

    def drain(semX, n=BLK):
        for _ in range(n):
            pltpu.make_async_copy(g_ref.at[pl.ds(0, 128)],
                                  bounce_v.at[pl.ds(0, 128)], semX).wait()

    load_start(row0, col0, ld0, 0)
    _zero_shared_slab(bounce_v, acc_sh, s)
    pltpu.sync_copy(g_ref, g_v)
    plsc.subcore_barrier()

    def pair(g, _):
        e0 = 2 * g
        load_wait(row0, col0, ld0)
        load_start_row(row1, ld1, e0 + 1)
        compute(row0, vals0)
        fire(col0, vals0, sc0)
        @pl.when(g >= 1)
        def _():
            drain(sc1)
        load_start_col(col1, ld1, e0 + 1)
        load_wait(row1, col1, ld1)
        load_start_row(row0, ld0, e0 + 2)
        compute(row1, vals1)
        fire(col1, vals1, sc1)
        drain(sc0)
        load_start_col(col0, ld0, e0 + 2)
        return 0

    lax.fori_loop(0, (NBLK - 1) // 2, pair, 0)

    load_wait(row0, col0, ld0)
    compute(row0, vals0)
    fire(col0, vals0, sc0)
    drain(sc1)

    @pl.when(wid < OCT_REM)
    def _extra_octet():
        toff = (row_base + NBLK * BLK) * 128
        pltpu.sync_copy(edge_ref.at[0, pl.ds(toff, 8 * 128)],
                        row1.at[pl.ds(0, 8 * 128)])
        pltpu.sync_copy(edge_ref.at[1, pl.ds(toff, 8 * 128)],
                        col1.at[pl.ds(0, 8 * 128)])
        compute(row1, vals1, n=8)
        fire(col1, vals1, sc1, n=8)
        drain(sc1, n=8)

    drain(sc0)
    plsc.subcore_barrier()
    _readout_shared_slab(acc_sh, bounce_v, out_hbm, c, s)


def _mid_body(deg_ref, x_ref, dinv_ref, g_ref):
    d = jnp.sum(deg_ref[...], axis=0) + 1.0
    dinv = lax.rsqrt(d)
    dinv_ref[...] = dinv
    g_ref[...] = dinv * x_ref[...]


def _fin_body(o_ref, dinv_ref, g_ref, w_ref, b_ref, out_ref):
    ssum = o_ref[0] + o_ref[1] + g_ref[...]
    z = w_ref[0, 0] * (dinv_ref[...] * ssum) + b_ref[0, 0]
    out_ref[...] = jax.nn.sigmoid(z)


def kernel(x, edge_index, W, b):
    e = edge_index.astype(jnp.int32)
    xp = jnp.pad(x.reshape(-1), (0, NP - N_NODES)).reshape(784, 128)

    deg_parts = _deg_kernel(e)

    dinv, g = pl.pallas_call(
        _mid_body,
        out_shape=[
            jax.ShapeDtypeStruct((784, 128), jnp.float32),
            jax.ShapeDtypeStruct((784, 128), jnp.float32),
        ],
    )(deg_parts.reshape(NW, 784, 128), xp)

    out_parts = _msg_kernel(e, g.reshape(NP)[:N_NODES])

    fin = pl.pallas_call(
        _fin_body,
        out_shape=jax.ShapeDtypeStruct((784, 128), jnp.float32),
    )(
        out_parts.reshape(2, 784, 128),
        dinv,
        g,
        W.astype(jnp.float32).reshape(1, 1),
        b.astype(jnp.float32).reshape(1, 1),
    )

    return fin.reshape(NP, 1)[:N_NODES]
